# baseline (speedup 1.0000x reference)
import functools

import jax
import jax.numpy as jnp
from jax import lax
from jax.experimental import pallas as pl
from jax.experimental.pallas import tpu as pltpu
from jax.experimental.pallas import tpu_sc as plsc

E = 64
TOPK = 2
H = 768
T = 2048
B = 16

GATE_TBLK = 256
MM_TBLK = 128
BG = 8
EPAD = 128


def _gate_body(x_ref, gw_ref, logits_ref, acc_ref):
    i = pl.program_id(0)

    @pl.when(i == 0)
    def _():
        acc_ref[...] = jnp.zeros_like(acc_ref)

    acc_ref[...] += jnp.sum(x_ref[...], axis=0)

    @pl.when(i == pl.num_programs(0) - 1)
    def _():
        pooled = acc_ref[...] * (1.0 / T)
        logits = lax.dot_general(
            pooled, gw_ref[...], (((1,), (1,)), ((), ())),
            preferred_element_type=jnp.float32)
        logits_ref[...] = jnp.concatenate(
            [logits, jnp.full((B, EPAD - E), -3e38, jnp.float32)], axis=1)


def _routing(inputs, gate_w):
    n_blk = T // GATE_TBLK
    return pl.pallas_call(
        _gate_body,
        grid=(n_blk,),
        in_specs=[
            pl.BlockSpec((GATE_TBLK, B, H), lambda i: (i, 0, 0)),
            pl.BlockSpec((E, H), lambda i: (0, 0)),
        ],
        out_specs=pl.BlockSpec((B, EPAD), lambda i: (0, 0)),
        out_shape=jax.ShapeDtypeStruct((B, EPAD), jnp.float32),
        scratch_shapes=[pltpu.VMEM((B, H), jnp.float32)],
    )(inputs, gate_w)


def _sc_select(logits_pad):
    mesh = plsc.VectorSubcoreMesh(core_axis_name="c", subcore_axis_name="s")

    @functools.partial(
        pl.kernel,
        mesh=mesh,
        out_type=[
            jax.ShapeDtypeStruct((B * 8,), jnp.int32),
            jax.ShapeDtypeStruct((B * 8,), jnp.float32),
        ],
        scratch_types=[
            pltpu.VMEM((EPAD,), jnp.float32),
            pltpu.VMEM((16,), jnp.int32),
            pltpu.VMEM((16,), jnp.float32),
        ],
    )
    def k(lg_hbm, sel_out, wts_out, lvec, sel_v, wts_v):
        c = lax.axis_index("c")
        s = lax.axis_index("s")
        b = c * (B // 2) + s

        @pl.when(s < B // 2)
        def _():
            pltpu.sync_copy(lg_hbm.at[b], lvec)
            lanes = lax.broadcasted_iota(jnp.int32, (16,), 0)
            xs = []
            for cch in range(E // 16):
                chv = lvec[pl.ds(cch * 16, 16)]
                xs.extend(chv[j] for j in range(16))
            m1 = xs[0]
            idx1 = jnp.int32(0)
            for j in range(1, E):
                better = xs[j] > m1
                m1 = jnp.where(better, xs[j], m1)
                idx1 = jnp.where(better, jnp.int32(j), idx1)
            m2 = jnp.float32(-3e38)
            idx2 = jnp.int32(0)
            for j in range(E):
                better = jnp.logical_and(xs[j] > m2, jnp.int32(j) != idx1)
                m2 = jnp.where(better, xs[j], m2)
                idx2 = jnp.where(better, jnp.int32(j), idx2)
            d = jnp.exp(jnp.full((16,), m2 - m1, jnp.float32))
            w1 = 1.0 / (1.0 + d)
            w2 = d / (1.0 + d)
            sel_v[...] = jnp.where(lanes == 0, idx1,
                                   jnp.where(lanes == 1, idx2, 0))
            wts_v[...] = jnp.where(lanes == 0, w1,
                                   jnp.where(lanes == 1, w2, 0.0))
            pltpu.sync_copy(sel_v.at[pl.ds(0, 8)], sel_out.at[pl.ds(b * 8, 8)])
            pltpu.sync_copy(wts_v.at[pl.ds(0, 8)], wts_out.at[pl.ds(b * 8, 8)])

    return k(logits_pad)


def _mm_body(sel_ref, wts_ref, x_ref, ew_ref, eb_ref, out_ref,
             wc0_ref, wc1_ref, bias_ref, stage_ref, sem_ref):
    g = pl.program_id(0)
    t = pl.program_id(1)

    def _issue(wc, slot, grp, i):
        b = grp * BG + i
        pltpu.make_async_copy(
            ew_ref.at[sel_ref[b * 8]], wc.at[i], sem_ref.at[slot, 0],
        ).start()
        pltpu.make_async_copy(
            ew_ref.at[sel_ref[b * 8 + 1]], stage_ref.at[slot],
            sem_ref.at[slot, 1],
        ).start()

    def _wait_combine(wc, slot, grp, i):
        b = grp * BG + i
        pltpu.make_async_copy(
            ew_ref.at[sel_ref[b * 8]], wc.at[i], sem_ref.at[slot, 0],
        ).wait()
        pltpu.make_async_copy(
            ew_ref.at[sel_ref[b * 8 + 1]], stage_ref.at[slot],
            sem_ref.at[slot, 1],
        ).wait()
        w0 = wts_ref[b * 8]
        w1 = wts_ref[b * 8 + 1]
        wc[i] = w0 * wc[i] + w1 * stage_ref[slot]

    @pl.when(jnp.logical_and(g == 0, t == 0))
    def _():
        for b in range(B):
            w0 = wts_ref[b * 8]
            w1 = wts_ref[b * 8 + 1]
            bias_ref[pl.ds(b, 1), :] = (
                w0 * eb_ref[pl.ds(sel_ref[b * 8], 1), :]
                + w1 * eb_ref[pl.ds(sel_ref[b * 8 + 1], 1), :])
        _issue(wc0_ref, 0, 0, 0)
        for i in range(BG):
            if i + 1 < BG:
                _issue(wc0_ref, (i + 1) % 2, 0, i + 1)
            _wait_combine(wc0_ref, i % 2, 0, i)
            y = lax.dot_general(
                x_ref[:, i, :], wc0_ref[i], (((1,), (1,)), ((), ())),
                preferred_element_type=jnp.float32)
            out_ref[:, i, :] = y + bias_ref[i][None, :]

    for j in range(BG):
        @pl.when(jnp.logical_and(g == 0, t == j + 1))
        def _(j=j):
            _issue(wc1_ref, j % 2, 1, j)

        @pl.when(jnp.logical_and(g == 0, t == j + 2))
        def _(j=j):
            _wait_combine(wc1_ref, j % 2, 1, j)

    @pl.when(jnp.logical_and(g == 0, t > 0))
    def _():
        for i in range(BG):
            y = lax.dot_general(
                x_ref[:, i, :], wc0_ref[i], (((1,), (1,)), ((), ())),
                preferred_element_type=jnp.float32)
            out_ref[:, i, :] = y + bias_ref[i][None, :]

    @pl.when(g == 1)
    def _():
        for i in range(BG):
            y = lax.dot_general(
                x_ref[:, i, :], wc1_ref[i], (((1,), (1,)), ((), ())),
                preferred_element_type=jnp.float32)
            out_ref[:, i, :] = y + bias_ref[BG + i][None, :]


def _expert_mm(inputs, expert_w, expert_b, sel, wts):
    n_t = T // MM_TBLK
    grid_spec = pltpu.PrefetchScalarGridSpec(
        num_scalar_prefetch=2,
        grid=(B // BG, n_t),
        in_specs=[
            pl.BlockSpec((MM_TBLK, BG, H), lambda g, t, sel, wts: (t, g, 0)),
            pl.BlockSpec(memory_space=pl.ANY),
            pl.BlockSpec((E, H), lambda g, t, sel, wts: (0, 0)),
        ],
        out_specs=pl.BlockSpec((MM_TBLK, BG, H), lambda g, t, sel, wts: (t, g, 0)),
        scratch_shapes=[
            pltpu.VMEM((BG, H, H), jnp.float32),
            pltpu.VMEM((BG, H, H), jnp.float32),
            pltpu.VMEM((B, H), jnp.float32),
            pltpu.VMEM((2, H, H), jnp.float32),
            pltpu.SemaphoreType.DMA((2, 2)),
        ],
    )
    return pl.pallas_call(
        _mm_body,
        grid_spec=grid_spec,
        out_shape=jax.ShapeDtypeStruct((T, B, H), jnp.float32),
        compiler_params=pltpu.CompilerParams(
            dimension_semantics=("arbitrary", "arbitrary")),
    )(sel, wts, inputs, expert_w, expert_b)


@jax.jit
def kernel(inputs, gate_w, expert_w, expert_b):
    logits_pad = _routing(inputs, gate_w)
    sel, wts = _sc_select(logits_pad)
    return _expert_mm(inputs, expert_w, expert_b, sel, wts)


# speedup vs baseline: 1.0712x; 1.0712x over previous
import functools

import jax
import jax.numpy as jnp
from jax import lax
from jax.experimental import pallas as pl
from jax.experimental.pallas import tpu as pltpu
from jax.experimental.pallas import tpu_sc as plsc

E = 64
TOPK = 2
H = 768
T = 2048
B = 16

GATE_TBLK = 256
MM_TBLK = 128
BG = 8
EPAD = 128


def _gate_body(x_ref, gw_ref, logits_ref, acc_ref):
    i = pl.program_id(0)

    @pl.when(i == 0)
    def _():
        acc_ref[...] = jnp.zeros_like(acc_ref)

    acc_ref[...] += jnp.sum(x_ref[...], axis=0)

    @pl.when(i == pl.num_programs(0) - 1)
    def _():
        pooled = acc_ref[...] * (1.0 / T)
        logits = lax.dot_general(
            pooled, gw_ref[...], (((1,), (1,)), ((), ())),
            preferred_element_type=jnp.float32)
        logits_ref[...] = jnp.concatenate(
            [logits, jnp.full((B, EPAD - E), -3e38, jnp.float32)], axis=1)


def _routing(inputs, gate_w):
    n_blk = T // GATE_TBLK
    return pl.pallas_call(
        _gate_body,
        grid=(n_blk,),
        in_specs=[
            pl.BlockSpec((GATE_TBLK, B, H), lambda i: (i, 0, 0)),
            pl.BlockSpec((E, H), lambda i: (0, 0)),
        ],
        out_specs=pl.BlockSpec((B, EPAD), lambda i: (0, 0)),
        out_shape=jax.ShapeDtypeStruct((B, EPAD), jnp.float32),
        scratch_shapes=[pltpu.VMEM((B, H), jnp.float32)],
    )(inputs, gate_w)


def _sc_select(logits_pad):
    mesh = plsc.VectorSubcoreMesh(core_axis_name="c", subcore_axis_name="s")

    @functools.partial(
        pl.kernel,
        mesh=mesh,
        out_type=[
            jax.ShapeDtypeStruct((B * 8,), jnp.int32),
            jax.ShapeDtypeStruct((B * 8,), jnp.float32),
        ],
        scratch_types=[
            pltpu.VMEM((EPAD,), jnp.float32),
            pltpu.VMEM((16,), jnp.int32),
            pltpu.VMEM((16,), jnp.float32),
        ],
    )
    def k(lg_hbm, sel_out, wts_out, lvec, sel_v, wts_v):
        c = lax.axis_index("c")
        s = lax.axis_index("s")
        b = c * (B // 2) + s

        @pl.when(s < B // 2)
        def _():
            pltpu.sync_copy(lg_hbm.at[b], lvec)
            lanes = lax.broadcasted_iota(jnp.int32, (16,), 0)
            xs = []
            for cch in range(E // 16):
                chv = lvec[pl.ds(cch * 16, 16)]
                xs.extend(chv[j] for j in range(16))
            m1 = xs[0]
            idx1 = jnp.int32(0)
            for j in range(1, E):
                better = xs[j] > m1
                m1 = jnp.where(better, xs[j], m1)
                idx1 = jnp.where(better, jnp.int32(j), idx1)
            m2 = jnp.float32(-3e38)
            idx2 = jnp.int32(0)
            for j in range(E):
                better = jnp.logical_and(xs[j] > m2, jnp.int32(j) != idx1)
                m2 = jnp.where(better, xs[j], m2)
                idx2 = jnp.where(better, jnp.int32(j), idx2)
            d = jnp.exp(jnp.full((16,), m2 - m1, jnp.float32))
            w1 = 1.0 / (1.0 + d)
            w2 = d / (1.0 + d)
            sel_v[...] = jnp.where(lanes == 0, idx1,
                                   jnp.where(lanes == 1, idx2, 0))
            wts_v[...] = jnp.where(lanes == 0, w1,
                                   jnp.where(lanes == 1, w2, 0.0))
            pltpu.sync_copy(sel_v.at[pl.ds(0, 8)], sel_out.at[pl.ds(b * 8, 8)])
            pltpu.sync_copy(wts_v.at[pl.ds(0, 8)], wts_out.at[pl.ds(b * 8, 8)])

    return k(logits_pad)


def _mm_body(sel_ref, wts_ref, x_ref, ew_ref, eb_ref, out_ref,
             wc0_ref, wc1_ref, bias_ref, stage_ref, sem_ref):
    g = pl.program_id(0)
    t = pl.program_id(1)

    def _issue(slot, grp, i):
        b = grp * BG + i
        for k in range(TOPK):
            pltpu.make_async_copy(
                ew_ref.at[sel_ref[b * 8 + k]], stage_ref.at[slot, k],
                sem_ref.at[slot, k],
            ).start()

    def _wait_combine(wc, slot, grp, i):
        b = grp * BG + i
        for k in range(TOPK):
            pltpu.make_async_copy(
                ew_ref.at[sel_ref[b * 8 + k]], stage_ref.at[slot, k],
                sem_ref.at[slot, k],
            ).wait()
        w0 = wts_ref[b * 8]
        w1 = wts_ref[b * 8 + 1]
        wc[i] = (w0 * stage_ref[slot, 0]
                 + w1 * stage_ref[slot, 1]).astype(jnp.bfloat16)

    @pl.when(jnp.logical_and(g == 0, t == 0))
    def _():
        for b in range(B):
            w0 = wts_ref[b * 8]
            w1 = wts_ref[b * 8 + 1]
            bias_ref[pl.ds(b, 1), :] = (
                w0 * eb_ref[pl.ds(sel_ref[b * 8], 1), :]
                + w1 * eb_ref[pl.ds(sel_ref[b * 8 + 1], 1), :])
        _issue(0, 0, 0)
        xb0 = x_ref[...].astype(jnp.bfloat16)
        for i in range(BG):
            if i + 1 < BG:
                _issue((i + 1) % 2, 0, i + 1)
            _wait_combine(wc0_ref, i % 2, 0, i)
            y = lax.dot_general(
                xb0[:, i, :], wc0_ref[i], (((1,), (1,)), ((), ())),
                preferred_element_type=jnp.float32)
            out_ref[:, i, :] = y + bias_ref[i][None, :]

    for j in range(BG):
        @pl.when(jnp.logical_and(g == 0, t == j + 1))
        def _(j=j):
            _issue(j % 2, 1, j)

        @pl.when(jnp.logical_and(g == 0, t == j + 2))
        def _(j=j):
            _wait_combine(wc1_ref, j % 2, 1, j)

    @pl.when(jnp.logical_and(g == 0, t > 0))
    def _():
        xb = x_ref[...].astype(jnp.bfloat16)
        for i in range(BG):
            y = lax.dot_general(
                xb[:, i, :], wc0_ref[i], (((1,), (1,)), ((), ())),
                preferred_element_type=jnp.float32)
            out_ref[:, i, :] = y + bias_ref[i][None, :]

    @pl.when(g == 1)
    def _():
        xb = x_ref[...].astype(jnp.bfloat16)
        for i in range(BG):
            y = lax.dot_general(
                xb[:, i, :], wc1_ref[i], (((1,), (1,)), ((), ())),
                preferred_element_type=jnp.float32)
            out_ref[:, i, :] = y + bias_ref[BG + i][None, :]


def _expert_mm(inputs, expert_w, expert_b, sel, wts):
    n_t = T // MM_TBLK
    grid_spec = pltpu.PrefetchScalarGridSpec(
        num_scalar_prefetch=2,
        grid=(B // BG, n_t),
        in_specs=[
            pl.BlockSpec((MM_TBLK, BG, H), lambda g, t, sel, wts: (t, g, 0)),
            pl.BlockSpec(memory_space=pl.ANY),
            pl.BlockSpec((E, H), lambda g, t, sel, wts: (0, 0)),
        ],
        out_specs=pl.BlockSpec((MM_TBLK, BG, H), lambda g, t, sel, wts: (t, g, 0)),
        scratch_shapes=[
            pltpu.VMEM((BG, H, H), jnp.bfloat16),
            pltpu.VMEM((BG, H, H), jnp.bfloat16),
            pltpu.VMEM((B, H), jnp.float32),
            pltpu.VMEM((2, TOPK, H, H), jnp.float32),
            pltpu.SemaphoreType.DMA((2, TOPK)),
        ],
    )
    return pl.pallas_call(
        _mm_body,
        grid_spec=grid_spec,
        out_shape=jax.ShapeDtypeStruct((T, B, H), jnp.float32),
        compiler_params=pltpu.CompilerParams(
            dimension_semantics=("arbitrary", "arbitrary")),
    )(sel, wts, inputs, expert_w, expert_b)


@jax.jit
def kernel(inputs, gate_w, expert_w, expert_b):
    logits_pad = _routing(inputs, gate_w)
    sel, wts = _sc_select(logits_pad)
    return _expert_mm(inputs, expert_w, expert_b, sel, wts)
